# Initial kernel scaffold; baseline (speedup 1.0000x reference)
import jax
import jax.numpy as jnp
from jax.experimental import pallas as pl


def kernel(x, edge_index, W_xz, b_xz, W_hz, b_hz, W_xr, b_xr, W_hr, b_hr, W_xh, b_xh, W_hh, b_hh, W_lin, b_lin):
    raise NotImplementedError("write your pallas kernel here")


# speedup vs baseline: 19.7832x; 19.7832x over previous
import functools

import jax
import jax.numpy as jnp
from jax import lax
from jax.experimental import pallas as pl
from jax.experimental.pallas import tpu as pltpu
from jax.experimental.pallas import tpu_sc as plsc

N = 10000
F_IN = 128
HID = 64
OUT = 16

NC = 2
NS = 16
NW = NC * NS
CH = 128

N_PAD = 10240
ROWS_PT = N_PAD // NS
DUMMY = N

BLK = 1000


def _deg_body(didx_hbm, ones_hbm, zeros_hbm, out_hbm, idx_v, idx_cur, ones_v,
              acc_sh, sem):
    c = lax.axis_index("c")
    s = lax.axis_index("s")
    w = c * NS + s
    cpw = didx_hbm.shape[1]
    pltpu.sync_copy(didx_hbm.at[w], idx_v)
    pltpu.sync_copy(ones_hbm, ones_v)
    pltpu.sync_copy(zeros_hbm, acc_sh.at[pl.ds(s * ROWS_PT, ROWS_PT)])
    plsc.subcore_barrier()

    def body(j, carry):
        for g in range(CH // 16):
            idx_cur[pl.ds(g * 16, 16)] = idx_v[j, pl.ds(g * 16, 16)]
        pltpu.sync_copy(ones_v, acc_sh.at[idx_cur], add=True)
        return carry

    lax.fori_loop(0, cpw, body, 0)
    plsc.subcore_barrier()
    pltpu.sync_copy(acc_sh.at[pl.ds(s * ROWS_PT, ROWS_PT)],
                    out_hbm.at[c, pl.ds(s * ROWS_PT, ROWS_PT)])


def _agg_body(y_hbm, gidx_hbm, sidx_hbm, zeros_hbm, out_hbm,
              gidx_v, sidx_v, gcur, scur, rows_v, acc_sh, sem):
    c = lax.axis_index("c")
    s = lax.axis_index("s")
    w = c * NS + s
    cpw = gidx_hbm.shape[1]
    pltpu.sync_copy(gidx_hbm.at[w], gidx_v)
    pltpu.sync_copy(sidx_hbm.at[w], sidx_v)
    pltpu.sync_copy(zeros_hbm, acc_sh.at[pl.ds(s * ROWS_PT, ROWS_PT)])
    plsc.subcore_barrier()

    def body(j, carry):
        for g in range(CH // 16):
            gcur[pl.ds(g * 16, 16)] = gidx_v[j, pl.ds(g * 16, 16)]
            scur[pl.ds(g * 16, 16)] = sidx_v[j, pl.ds(g * 16, 16)]
        pltpu.async_copy(y_hbm.at[gcur], rows_v, sem).wait()
        pltpu.sync_copy(rows_v, acc_sh.at[scur], add=True)
        return carry

    lax.fori_loop(0, cpw, body, 0)
    plsc.subcore_barrier()
    pltpu.sync_copy(acc_sh.at[pl.ds(s * ROWS_PT, ROWS_PT)],
                    out_hbm.at[c, pl.ds(s * ROWS_PT, ROWS_PT)])


def _prep_body(x_ref, degp_ref, wcat_ref, wz0_ref, wh0_ref, bz_ref, bh_ref,
               y_ref, xz_ref, xh_ref, dinv_ref):
    deg = degp_ref[0, :, 0:1] + degp_ref[1, :, 0:1]
    dinv = jnp.where(deg > 0, lax.rsqrt(jnp.maximum(deg, 1e-12)), 0.0)
    xb = x_ref[...]
    y_ref[...] = jnp.dot(xb * dinv, wcat_ref[...],
                         preferred_element_type=jnp.float32)
    xz_ref[...] = jnp.dot(xb, wz0_ref[...],
                          preferred_element_type=jnp.float32) + bz_ref[...]
    xh_ref[...] = jnp.dot(xb, wh0_ref[...],
                          preferred_element_type=jnp.float32) + bh_ref[...]
    dinv_ref[...] = dinv


def _final_body(aggp_ref, dinv_ref, xz_ref, xh_ref, wlin_ref, blin_ref, out_ref):
    agg = aggp_ref[0] + aggp_ref[1]
    t = agg * (-dinv_ref[...])
    z = jax.nn.sigmoid(xz_ref[...] + t[:, :HID])
    ht = jnp.tanh(xh_ref[...] + t[:, HID:])
    h = jnp.dot((1.0 - z) * ht, wlin_ref[...],
                preferred_element_type=jnp.float32) + blin_ref[...]
    nrm = jnp.maximum(jnp.sqrt(jnp.sum(h * h, axis=1, keepdims=True)), 1e-12)
    out_ref[...] = h / nrm


def kernel(x, edge_index, W_xz, b_xz, W_hz, b_hz, W_xr, b_xr, W_hr, b_hr,
           W_xh, b_xh, W_hh, b_hh, W_lin, b_lin):
    e = edge_index.shape[1]
    cpw = -(-e // (NW * CH))
    e_pad = NW * cpw * CH
    pad = e_pad - e

    src = edge_index[0]
    dst = edge_index[1]
    deg_idx = jnp.concatenate(
        [src, jnp.full((pad,), DUMMY, jnp.int32)]).reshape(NW, cpw, CH)
    gat_idx = jnp.concatenate(
        [src, jnp.zeros((pad,), jnp.int32)]).reshape(NW, cpw, CH)
    sct_idx = jnp.concatenate(
        [dst, jnp.full((pad,), DUMMY, jnp.int32)]).reshape(NW, cpw, CH)

    ones_rows = jnp.ones((CH, F_IN), jnp.float32)
    zeros_agg = jnp.zeros((ROWS_PT, F_IN), jnp.float32)

    mesh = plsc.VectorSubcoreMesh(core_axis_name="c", subcore_axis_name="s")

    deg_call = pl.kernel(
        _deg_body,
        out_type=jax.ShapeDtypeStruct((NC, N_PAD, F_IN), jnp.float32),
        mesh=mesh,
        scratch_types=[
            pltpu.VMEM((cpw, CH), jnp.int32),
            pltpu.VMEM((CH,), jnp.int32),
            pltpu.VMEM((CH, F_IN), jnp.float32),
            pltpu.VMEM_SHARED((N_PAD, F_IN), jnp.float32),
            pltpu.SemaphoreType.DMA,
        ],
    )
    degp = deg_call(deg_idx, ones_rows, zeros_agg)

    wcat = jnp.concatenate([W_xz[1], W_xh[1]], axis=1)
    bz = (b_xz + b_hz).reshape(1, HID)
    bh = (b_xh + b_hh).reshape(1, HID)

    nblk = N // BLK
    y, xz, xh, dinv = pl.pallas_call(
        _prep_body,
        grid=(nblk,),
        in_specs=[
            pl.BlockSpec((BLK, F_IN), lambda i: (i, 0)),
            pl.BlockSpec((NC, BLK, F_IN), lambda i: (0, i, 0)),
            pl.BlockSpec((F_IN, F_IN), lambda i: (0, 0)),
            pl.BlockSpec((F_IN, HID), lambda i: (0, 0)),
            pl.BlockSpec((F_IN, HID), lambda i: (0, 0)),
            pl.BlockSpec((1, HID), lambda i: (0, 0)),
            pl.BlockSpec((1, HID), lambda i: (0, 0)),
        ],
        out_specs=[
            pl.BlockSpec((BLK, F_IN), lambda i: (i, 0)),
            pl.BlockSpec((BLK, HID), lambda i: (i, 0)),
            pl.BlockSpec((BLK, HID), lambda i: (i, 0)),
            pl.BlockSpec((BLK, 1), lambda i: (i, 0)),
        ],
        out_shape=[
            jax.ShapeDtypeStruct((N, F_IN), jnp.float32),
            jax.ShapeDtypeStruct((N, HID), jnp.float32),
            jax.ShapeDtypeStruct((N, HID), jnp.float32),
            jax.ShapeDtypeStruct((N, 1), jnp.float32),
        ],
    )(x, degp, wcat, W_xz[0], W_xh[0], bz, bh)

    agg_call = pl.kernel(
        _agg_body,
        out_type=jax.ShapeDtypeStruct((NC, N_PAD, F_IN), jnp.float32),
        mesh=mesh,
        scratch_types=[
            pltpu.VMEM((cpw, CH), jnp.int32),
            pltpu.VMEM((cpw, CH), jnp.int32),
            pltpu.VMEM((CH,), jnp.int32),
            pltpu.VMEM((CH,), jnp.int32),
            pltpu.VMEM((CH, F_IN), jnp.float32),
            pltpu.VMEM_SHARED((N_PAD, F_IN), jnp.float32),
            pltpu.SemaphoreType.DMA,
        ],
    )
    aggp = agg_call(y, gat_idx, sct_idx, zeros_agg)

    out = pl.pallas_call(
        _final_body,
        grid=(nblk,),
        in_specs=[
            pl.BlockSpec((NC, BLK, F_IN), lambda i: (0, i, 0)),
            pl.BlockSpec((BLK, 1), lambda i: (i, 0)),
            pl.BlockSpec((BLK, HID), lambda i: (i, 0)),
            pl.BlockSpec((BLK, HID), lambda i: (i, 0)),
            pl.BlockSpec((HID, OUT), lambda i: (0, 0)),
            pl.BlockSpec((1, OUT), lambda i: (0, 0)),
        ],
        out_specs=pl.BlockSpec((BLK, OUT), lambda i: (i, 0)),
        out_shape=jax.ShapeDtypeStruct((N, OUT), jnp.float32),
    )(aggp, dinv, xz, xh, W_lin, b_lin.reshape(1, OUT))
    return out


# speedup vs baseline: 21.4724x; 1.0854x over previous
import functools

import jax
import jax.numpy as jnp
from jax import lax
from jax.experimental import pallas as pl
from jax.experimental.pallas import tpu as pltpu
from jax.experimental.pallas import tpu_sc as plsc

N = 10000
F_IN = 128
HID = 64
OUT = 16

NC = 2
NS = 16
NW = NC * NS
CH = 128

N_PAD = 10240
ROWS_PT = N_PAD // NS
DUMMY = N

BLK = 1000


def _deg_body(didx_hbm, ones_hbm, zeros_hbm, out_hbm, idx_v, idx_cur, ones_v,
              acc_sh, sem):
    c = lax.axis_index("c")
    s = lax.axis_index("s")
    w = c * NS + s
    cpw = didx_hbm.shape[1]
    pltpu.sync_copy(didx_hbm.at[w], idx_v)
    pltpu.sync_copy(ones_hbm, ones_v)
    pltpu.sync_copy(zeros_hbm, acc_sh.at[pl.ds(s * ROWS_PT, ROWS_PT)])
    plsc.subcore_barrier()

    def body(j, carry):
        for g in range(CH // 16):
            idx_cur[pl.ds(g * 16, 16)] = idx_v[j, pl.ds(g * 16, 16)]
        pltpu.sync_copy(ones_v, acc_sh.at[idx_cur], add=True)
        return carry

    lax.fori_loop(0, cpw, body, 0)
    plsc.subcore_barrier()
    pltpu.sync_copy(acc_sh.at[pl.ds(s * ROWS_PT, ROWS_PT)],
                    out_hbm.at[c, pl.ds(s * ROWS_PT, ROWS_PT)])


def _agg_body(y_hbm, gidx_hbm, sidx_hbm, zeros_hbm, out_hbm,
              gcur0, scur0, gcur1, scur1, rows0, rows1,
              acc_sh, semA, semB, semI0, semI1):
    c = lax.axis_index("c")
    s = lax.axis_index("s")
    w = c * NS + s
    cpw = gidx_hbm.shape[1]
    pltpu.sync_copy(zeros_hbm, acc_sh.at[pl.ds(s * ROWS_PT, ROWS_PT)])

    def fetch(j, gbuf, sbuf, sem):
        pltpu.async_copy(gidx_hbm.at[w, j], gbuf, sem)
        pltpu.async_copy(sidx_hbm.at[w, j], sbuf, sem)

    def wait_fetch(gbuf, sbuf, sem):
        pltpu.make_async_copy(gidx_hbm.at[w, 0], gbuf, sem).wait()
        pltpu.make_async_copy(sidx_hbm.at[w, 0], sbuf, sem).wait()

    plsc.subcore_barrier()

    fetch(0, gcur0, scur0, semI0)
    wait_fetch(gcur0, scur0, semI0)
    pltpu.async_copy(y_hbm.at[gcur0], rows0, semA)
    fetch(1, gcur1, scur1, semI1)

    def body(jj, carry):
        a = 2 * jj
        b = a + 1
        wait_fetch(gcur1, scur1, semI1)
        pltpu.async_copy(y_hbm.at[gcur1], rows1, semB)
        pltpu.make_async_copy(y_hbm.at[gcur0], rows0, semA).wait()
        pltpu.sync_copy(rows0, acc_sh.at[scur0], add=True)

        @pl.when(a + 2 < cpw)
        def _():
            fetch(a + 2, gcur0, scur0, semI0)

        pltpu.make_async_copy(y_hbm.at[gcur1], rows1, semB).wait()
        pltpu.sync_copy(rows1, acc_sh.at[scur1], add=True)

        @pl.when(b + 2 < cpw)
        def _():
            fetch(b + 2, gcur1, scur1, semI1)

        @pl.when(a + 2 < cpw)
        def _():
            wait_fetch(gcur0, scur0, semI0)
            pltpu.async_copy(y_hbm.at[gcur0], rows0, semA)

        return carry

    lax.fori_loop(0, cpw // 2, body, 0)
    if cpw % 2 == 1:
        pltpu.make_async_copy(y_hbm.at[gcur0], rows0, semA).wait()
        pltpu.sync_copy(rows0, acc_sh.at[scur0], add=True)
    plsc.subcore_barrier()
    pltpu.sync_copy(acc_sh.at[pl.ds(s * ROWS_PT, ROWS_PT)],
                    out_hbm.at[c, pl.ds(s * ROWS_PT, ROWS_PT)])


def _prep_body(x_ref, degp_ref, wcat_ref, wz0_ref, wh0_ref, bz_ref, bh_ref,
               y_ref, xz_ref, xh_ref, dinv_ref):
    deg = degp_ref[0, :, 0:1] + degp_ref[1, :, 0:1]
    dinv = jnp.where(deg > 0, lax.rsqrt(jnp.maximum(deg, 1e-12)), 0.0)
    xb = x_ref[...]
    y_ref[...] = jnp.dot(xb * dinv, wcat_ref[...],
                         preferred_element_type=jnp.float32)
    xz_ref[...] = jnp.dot(xb, wz0_ref[...],
                          preferred_element_type=jnp.float32) + bz_ref[...]
    xh_ref[...] = jnp.dot(xb, wh0_ref[...],
                          preferred_element_type=jnp.float32) + bh_ref[...]
    dinv_ref[...] = dinv


def _final_body(aggp_ref, dinv_ref, xz_ref, xh_ref, wlin_ref, blin_ref, out_ref):
    agg = aggp_ref[0] + aggp_ref[1]
    t = agg * (-dinv_ref[...])
    z = jax.nn.sigmoid(xz_ref[...] + t[:, :HID])
    ht = jnp.tanh(xh_ref[...] + t[:, HID:])
    h = jnp.dot((1.0 - z) * ht, wlin_ref[...],
                preferred_element_type=jnp.float32) + blin_ref[...]
    nrm = jnp.maximum(jnp.sqrt(jnp.sum(h * h, axis=1, keepdims=True)), 1e-12)
    out_ref[...] = h / nrm


def kernel(x, edge_index, W_xz, b_xz, W_hz, b_hz, W_xr, b_xr, W_hr, b_hr,
           W_xh, b_xh, W_hh, b_hh, W_lin, b_lin):
    e = edge_index.shape[1]
    cpw = -(-e // (NW * CH))
    e_pad = NW * cpw * CH
    pad = e_pad - e

    src = edge_index[0]
    dst = edge_index[1]
    deg_idx = jnp.concatenate(
        [src, jnp.full((pad,), DUMMY, jnp.int32)]).reshape(NW, cpw, CH)
    gat_idx = jnp.concatenate(
        [src, jnp.zeros((pad,), jnp.int32)]).reshape(NW, cpw, CH)
    sct_idx = jnp.concatenate(
        [dst, jnp.full((pad,), DUMMY, jnp.int32)]).reshape(NW, cpw, CH)

    ones_rows = jnp.ones((CH, F_IN), jnp.float32)
    zeros_agg = jnp.zeros((ROWS_PT, F_IN), jnp.float32)

    mesh = plsc.VectorSubcoreMesh(core_axis_name="c", subcore_axis_name="s")

    deg_call = pl.kernel(
        _deg_body,
        out_type=jax.ShapeDtypeStruct((NC, N_PAD, F_IN), jnp.float32),
        mesh=mesh,
        scratch_types=[
            pltpu.VMEM((cpw, CH), jnp.int32),
            pltpu.VMEM((CH,), jnp.int32),
            pltpu.VMEM((CH, F_IN), jnp.float32),
            pltpu.VMEM_SHARED((N_PAD, F_IN), jnp.float32),
            pltpu.SemaphoreType.DMA,
        ],
    )
    degp = deg_call(deg_idx, ones_rows, zeros_agg)

    wcat = jnp.concatenate([W_xz[1], W_xh[1]], axis=1)
    bz = (b_xz + b_hz).reshape(1, HID)
    bh = (b_xh + b_hh).reshape(1, HID)

    nblk = N // BLK
    y, xz, xh, dinv = pl.pallas_call(
        _prep_body,
        grid=(nblk,),
        in_specs=[
            pl.BlockSpec((BLK, F_IN), lambda i: (i, 0)),
            pl.BlockSpec((NC, BLK, F_IN), lambda i: (0, i, 0)),
            pl.BlockSpec((F_IN, F_IN), lambda i: (0, 0)),
            pl.BlockSpec((F_IN, HID), lambda i: (0, 0)),
            pl.BlockSpec((F_IN, HID), lambda i: (0, 0)),
            pl.BlockSpec((1, HID), lambda i: (0, 0)),
            pl.BlockSpec((1, HID), lambda i: (0, 0)),
        ],
        out_specs=[
            pl.BlockSpec((BLK, F_IN), lambda i: (i, 0)),
            pl.BlockSpec((BLK, HID), lambda i: (i, 0)),
            pl.BlockSpec((BLK, HID), lambda i: (i, 0)),
            pl.BlockSpec((BLK, 1), lambda i: (i, 0)),
        ],
        out_shape=[
            jax.ShapeDtypeStruct((N, F_IN), jnp.float32),
            jax.ShapeDtypeStruct((N, HID), jnp.float32),
            jax.ShapeDtypeStruct((N, HID), jnp.float32),
            jax.ShapeDtypeStruct((N, 1), jnp.float32),
        ],
    )(x, degp, wcat, W_xz[0], W_xh[0], bz, bh)

    agg_call = pl.kernel(
        _agg_body,
        out_type=jax.ShapeDtypeStruct((NC, N_PAD, F_IN), jnp.float32),
        mesh=mesh,
        scratch_types=[
            pltpu.VMEM((CH,), jnp.int32),
            pltpu.VMEM((CH,), jnp.int32),
            pltpu.VMEM((CH,), jnp.int32),
            pltpu.VMEM((CH,), jnp.int32),
            pltpu.VMEM((CH, F_IN), jnp.float32),
            pltpu.VMEM((CH, F_IN), jnp.float32),
            pltpu.VMEM_SHARED((N_PAD, F_IN), jnp.float32),
            pltpu.SemaphoreType.DMA,
            pltpu.SemaphoreType.DMA,
            pltpu.SemaphoreType.DMA,
            pltpu.SemaphoreType.DMA,
        ],
    )
    aggp = agg_call(y, gat_idx, sct_idx, zeros_agg)

    out = pl.pallas_call(
        _final_body,
        grid=(nblk,),
        in_specs=[
            pl.BlockSpec((NC, BLK, F_IN), lambda i: (0, i, 0)),
            pl.BlockSpec((BLK, 1), lambda i: (i, 0)),
            pl.BlockSpec((BLK, HID), lambda i: (i, 0)),
            pl.BlockSpec((BLK, HID), lambda i: (i, 0)),
            pl.BlockSpec((HID, OUT), lambda i: (0, 0)),
            pl.BlockSpec((1, OUT), lambda i: (0, 0)),
        ],
        out_specs=pl.BlockSpec((BLK, OUT), lambda i: (i, 0)),
        out_shape=jax.ShapeDtypeStruct((N, OUT), jnp.float32),
    )(aggp, dinv, xz, xh, W_lin, b_lin.reshape(1, OUT))
    return out


# speedup vs baseline: 22.5232x; 1.0489x over previous
import functools

import jax
import jax.numpy as jnp
from jax import lax
from jax.experimental import pallas as pl
from jax.experimental.pallas import tpu as pltpu
from jax.experimental.pallas import tpu_sc as plsc

N = 10000
F_IN = 128
HID = 64
OUT = 16

NC = 2
NS = 16
NW = NC * NS
CH = 128

N_PAD = 10240
ROWS_PT = N_PAD // NS
DUMMY = N

BLK = 1000


def _deg_body(didx_hbm, ones_hbm, zeros_hbm, out_hbm, idx_v, idx_cur, ones_v,
              acc_sh, sem):
    c = lax.axis_index("c")
    s = lax.axis_index("s")
    w = c * NS + s
    cpw = didx_hbm.shape[1]
    pltpu.sync_copy(didx_hbm.at[w], idx_v)
    pltpu.sync_copy(ones_hbm, ones_v)
    pltpu.sync_copy(zeros_hbm, acc_sh.at[pl.ds(s * ROWS_PT, ROWS_PT)])
    plsc.subcore_barrier()

    def body(j, carry):
        for g in range(CH // 16):
            idx_cur[pl.ds(g * 16, 16)] = idx_v[j, pl.ds(g * 16, 16)]
        pltpu.sync_copy(ones_v, acc_sh.at[idx_cur], add=True)
        return carry

    lax.fori_loop(0, cpw, body, 0)
    plsc.subcore_barrier()
    pltpu.sync_copy(acc_sh.at[pl.ds(s * ROWS_PT, ROWS_PT)],
                    out_hbm.at[c, pl.ds(s * ROWS_PT, ROWS_PT)])


def _agg_body(y_hbm, gidx_hbm, sidx_hbm, zeros_hbm, out_hbm,
              gcur0, scur0, gcur1, scur1, rows0, rows1,
              acc_sh, semA, semB, semI0, semI1):
    c = lax.axis_index("c")
    s = lax.axis_index("s")
    w = c * NS + s
    cpw = gidx_hbm.shape[1]
    pltpu.sync_copy(zeros_hbm, acc_sh.at[pl.ds(s * ROWS_PT, ROWS_PT)])

    def fetch(j, gbuf, sbuf, sem):
        pltpu.async_copy(gidx_hbm.at[w, j], gbuf, sem)
        pltpu.async_copy(sidx_hbm.at[w, j], sbuf, sem)

    def wait_fetch(gbuf, sbuf, sem):
        pltpu.make_async_copy(gidx_hbm.at[w, 0], gbuf, sem).wait()
        pltpu.make_async_copy(sidx_hbm.at[w, 0], sbuf, sem).wait()

    plsc.subcore_barrier()

    fetch(0, gcur0, scur0, semI0)
    wait_fetch(gcur0, scur0, semI0)
    pltpu.async_copy(y_hbm.at[gcur0], rows0, semA)
    fetch(1, gcur1, scur1, semI1)

    def body(jj, carry):
        a = 2 * jj
        b = a + 1
        wait_fetch(gcur1, scur1, semI1)
        pltpu.async_copy(y_hbm.at[gcur1], rows1, semB)
        pltpu.make_async_copy(y_hbm.at[gcur0], rows0, semA).wait()

        @pl.when(a + 2 < cpw)
        def _():
            pltpu.async_copy(gidx_hbm.at[w, a + 2], gcur0, semI0)

        pltpu.sync_copy(rows0, acc_sh.at[scur0], add=True)

        @pl.when(a + 2 < cpw)
        def _():
            pltpu.async_copy(sidx_hbm.at[w, a + 2], scur0, semI0)
            wait_fetch(gcur0, scur0, semI0)
            pltpu.async_copy(y_hbm.at[gcur0], rows0, semA)

        pltpu.make_async_copy(y_hbm.at[gcur1], rows1, semB).wait()
        pltpu.sync_copy(rows1, acc_sh.at[scur1], add=True)

        @pl.when(b + 2 < cpw)
        def _():
            fetch(b + 2, gcur1, scur1, semI1)

        return carry

    lax.fori_loop(0, cpw // 2, body, 0)
    if cpw % 2 == 1:
        pltpu.make_async_copy(y_hbm.at[gcur0], rows0, semA).wait()
        pltpu.sync_copy(rows0, acc_sh.at[scur0], add=True)
    plsc.subcore_barrier()
    pltpu.sync_copy(acc_sh.at[pl.ds(s * ROWS_PT, ROWS_PT)],
                    out_hbm.at[c, pl.ds(s * ROWS_PT, ROWS_PT)])


def _prep_body(x_ref, degp_ref, wcat_ref, wz0_ref, wh0_ref, bz_ref, bh_ref,
               y_ref, xz_ref, xh_ref, dinv_ref):
    deg = degp_ref[0, :, 0:1] + degp_ref[1, :, 0:1]
    dinv = jnp.where(deg > 0, lax.rsqrt(jnp.maximum(deg, 1e-12)), 0.0)
    xb = x_ref[...]
    y_ref[...] = jnp.dot(xb * dinv, wcat_ref[...],
                         preferred_element_type=jnp.float32)
    xz_ref[...] = jnp.dot(xb, wz0_ref[...],
                          preferred_element_type=jnp.float32) + bz_ref[...]
    xh_ref[...] = jnp.dot(xb, wh0_ref[...],
                          preferred_element_type=jnp.float32) + bh_ref[...]
    dinv_ref[...] = dinv


def _final_body(aggp_ref, dinv_ref, xz_ref, xh_ref, wlin_ref, blin_ref, out_ref):
    agg = aggp_ref[0] + aggp_ref[1]
    t = agg * (-dinv_ref[...])
    z = jax.nn.sigmoid(xz_ref[...] + t[:, :HID])
    ht = jnp.tanh(xh_ref[...] + t[:, HID:])
    h = jnp.dot((1.0 - z) * ht, wlin_ref[...],
                preferred_element_type=jnp.float32) + blin_ref[...]
    nrm = jnp.maximum(jnp.sqrt(jnp.sum(h * h, axis=1, keepdims=True)), 1e-12)
    out_ref[...] = h / nrm


def kernel(x, edge_index, W_xz, b_xz, W_hz, b_hz, W_xr, b_xr, W_hr, b_hr,
           W_xh, b_xh, W_hh, b_hh, W_lin, b_lin):
    e = edge_index.shape[1]
    cpw = -(-e // (NW * CH))
    e_pad = NW * cpw * CH
    pad = e_pad - e

    src = edge_index[0]
    dst = edge_index[1]
    deg_idx = jnp.concatenate(
        [src, jnp.full((pad,), DUMMY, jnp.int32)]).reshape(NW, cpw, CH)
    gat_idx = jnp.concatenate(
        [src, jnp.zeros((pad,), jnp.int32)]).reshape(NW, cpw, CH)
    sct_idx = jnp.concatenate(
        [dst, jnp.full((pad,), DUMMY, jnp.int32)]).reshape(NW, cpw, CH)

    ones_rows = jnp.ones((CH, F_IN), jnp.float32)
    zeros_agg = jnp.zeros((ROWS_PT, F_IN), jnp.float32)

    mesh = plsc.VectorSubcoreMesh(core_axis_name="c", subcore_axis_name="s")

    deg_call = pl.kernel(
        _deg_body,
        out_type=jax.ShapeDtypeStruct((NC, N_PAD, F_IN), jnp.float32),
        mesh=mesh,
        scratch_types=[
            pltpu.VMEM((cpw, CH), jnp.int32),
            pltpu.VMEM((CH,), jnp.int32),
            pltpu.VMEM((CH, F_IN), jnp.float32),
            pltpu.VMEM_SHARED((N_PAD, F_IN), jnp.float32),
            pltpu.SemaphoreType.DMA,
        ],
    )
    degp = deg_call(deg_idx, ones_rows, zeros_agg)

    wcat = jnp.concatenate([W_xz[1], W_xh[1]], axis=1)
    bz = (b_xz + b_hz).reshape(1, HID)
    bh = (b_xh + b_hh).reshape(1, HID)

    nblk = N // BLK
    y, xz, xh, dinv = pl.pallas_call(
        _prep_body,
        grid=(nblk,),
        in_specs=[
            pl.BlockSpec((BLK, F_IN), lambda i: (i, 0)),
            pl.BlockSpec((NC, BLK, F_IN), lambda i: (0, i, 0)),
            pl.BlockSpec((F_IN, F_IN), lambda i: (0, 0)),
            pl.BlockSpec((F_IN, HID), lambda i: (0, 0)),
            pl.BlockSpec((F_IN, HID), lambda i: (0, 0)),
            pl.BlockSpec((1, HID), lambda i: (0, 0)),
            pl.BlockSpec((1, HID), lambda i: (0, 0)),
        ],
        out_specs=[
            pl.BlockSpec((BLK, F_IN), lambda i: (i, 0)),
            pl.BlockSpec((BLK, HID), lambda i: (i, 0)),
            pl.BlockSpec((BLK, HID), lambda i: (i, 0)),
            pl.BlockSpec((BLK, 1), lambda i: (i, 0)),
        ],
        out_shape=[
            jax.ShapeDtypeStruct((N, F_IN), jnp.float32),
            jax.ShapeDtypeStruct((N, HID), jnp.float32),
            jax.ShapeDtypeStruct((N, HID), jnp.float32),
            jax.ShapeDtypeStruct((N, 1), jnp.float32),
        ],
    )(x, degp, wcat, W_xz[0], W_xh[0], bz, bh)

    agg_call = pl.kernel(
        _agg_body,
        out_type=jax.ShapeDtypeStruct((NC, N_PAD, F_IN), jnp.float32),
        mesh=mesh,
        scratch_types=[
            pltpu.VMEM((CH,), jnp.int32),
            pltpu.VMEM((CH,), jnp.int32),
            pltpu.VMEM((CH,), jnp.int32),
            pltpu.VMEM((CH,), jnp.int32),
            pltpu.VMEM((CH, F_IN), jnp.float32),
            pltpu.VMEM((CH, F_IN), jnp.float32),
            pltpu.VMEM_SHARED((N_PAD, F_IN), jnp.float32),
            pltpu.SemaphoreType.DMA,
            pltpu.SemaphoreType.DMA,
            pltpu.SemaphoreType.DMA,
            pltpu.SemaphoreType.DMA,
        ],
    )
    aggp = agg_call(y, gat_idx, sct_idx, zeros_agg)

    out = pl.pallas_call(
        _final_body,
        grid=(nblk,),
        in_specs=[
            pl.BlockSpec((NC, BLK, F_IN), lambda i: (0, i, 0)),
            pl.BlockSpec((BLK, 1), lambda i: (i, 0)),
            pl.BlockSpec((BLK, HID), lambda i: (i, 0)),
            pl.BlockSpec((BLK, HID), lambda i: (i, 0)),
            pl.BlockSpec((HID, OUT), lambda i: (0, 0)),
            pl.BlockSpec((1, OUT), lambda i: (0, 0)),
        ],
        out_specs=pl.BlockSpec((BLK, OUT), lambda i: (i, 0)),
        out_shape=jax.ShapeDtypeStruct((N, OUT), jnp.float32),
    )(aggp, dinv, xz, xh, W_lin, b_lin.reshape(1, OUT))
    return out


# speedup vs baseline: 24.8561x; 1.1036x over previous
import functools

import jax
import jax.numpy as jnp
from jax import lax
from jax.experimental import pallas as pl
from jax.experimental.pallas import tpu as pltpu
from jax.experimental.pallas import tpu_sc as plsc

N = 10000
F_IN = 128
HID = 64
OUT = 16

NC = 2
NS = 16
NW = NC * NS
CH = 128

N_PAD = 10240
ROWS_PT = N_PAD // NS
DUMMY = N

BLK = 1000

CPW_SC0 = 51
CPW_SC1 = 106
CPW_MAX = max(CPW_SC0, CPW_SC1)


def _deg_body(didx_hbm, ones_hbm, zeros_hbm, out_hbm, idx_v, idx_cur, ones_v,
              acc_sh, sem):
    c = lax.axis_index("c")
    s = lax.axis_index("s")
    w = c * NS + s
    cpw = didx_hbm.shape[1]
    pltpu.sync_copy(didx_hbm.at[w], idx_v)
    pltpu.sync_copy(ones_hbm, ones_v)
    pltpu.sync_copy(zeros_hbm, acc_sh.at[pl.ds(s * ROWS_PT, ROWS_PT)])
    plsc.subcore_barrier()

    def body(j, carry):
        for g in range(CH // 16):
            idx_cur[pl.ds(g * 16, 16)] = idx_v[j, pl.ds(g * 16, 16)]
        pltpu.sync_copy(ones_v, acc_sh.at[idx_cur], add=True)
        return carry

    lax.fori_loop(0, cpw, body, 0)
    plsc.subcore_barrier()
    pltpu.sync_copy(acc_sh.at[pl.ds(s * ROWS_PT, ROWS_PT)],
                    out_hbm.at[c, pl.ds(s * ROWS_PT, ROWS_PT)])


def _agg_body(y_hbm, gidx_hbm, sidx_hbm, zeros_hbm, out_hbm,
              gcur0, scur0, gcur1, scur1, rows0, rows1,
              acc_sh, semA, semB, semI0, semI1):
    c = lax.axis_index("c")
    s = lax.axis_index("s")
    w = c * NS + s
    cpw = jnp.where(c == 0, CPW_SC0, CPW_SC1)
    pltpu.sync_copy(zeros_hbm, acc_sh.at[pl.ds(s * ROWS_PT, ROWS_PT)])

    def fetch(j, gbuf, sbuf, sem):
        pltpu.async_copy(gidx_hbm.at[w, j], gbuf, sem)
        pltpu.async_copy(sidx_hbm.at[w, j], sbuf, sem)

    def wait_fetch(gbuf, sbuf, sem):
        pltpu.make_async_copy(gidx_hbm.at[w, 0], gbuf, sem).wait()
        pltpu.make_async_copy(sidx_hbm.at[w, 0], sbuf, sem).wait()

    plsc.subcore_barrier()

    fetch(0, gcur0, scur0, semI0)
    wait_fetch(gcur0, scur0, semI0)
    pltpu.async_copy(y_hbm.at[gcur0], rows0, semA)
    fetch(1, gcur1, scur1, semI1)

    def body(jj, carry):
        a = 2 * jj
        b = a + 1
        wait_fetch(gcur1, scur1, semI1)
        pltpu.async_copy(y_hbm.at[gcur1], rows1, semB)
        pltpu.make_async_copy(y_hbm.at[gcur0], rows0, semA).wait()

        @pl.when(a + 2 < cpw)
        def _():
            pltpu.async_copy(gidx_hbm.at[w, a + 2], gcur0, semI0)

        pltpu.sync_copy(rows0, acc_sh.at[scur0], add=True)

        @pl.when(a + 2 < cpw)
        def _():
            pltpu.async_copy(sidx_hbm.at[w, a + 2], scur0, semI0)
            wait_fetch(gcur0, scur0, semI0)
            pltpu.async_copy(y_hbm.at[gcur0], rows0, semA)

        pltpu.make_async_copy(y_hbm.at[gcur1], rows1, semB).wait()
        pltpu.sync_copy(rows1, acc_sh.at[scur1], add=True)

        @pl.when(b + 2 < cpw)
        def _():
            fetch(b + 2, gcur1, scur1, semI1)

        return carry

    lax.fori_loop(0, cpw // 2, body, 0)

    @pl.when(cpw % 2 == 1)
    def _():
        pltpu.make_async_copy(y_hbm.at[gcur0], rows0, semA).wait()
        pltpu.sync_copy(rows0, acc_sh.at[scur0], add=True)
    plsc.subcore_barrier()
    pltpu.sync_copy(acc_sh.at[pl.ds(s * ROWS_PT, ROWS_PT)],
                    out_hbm.at[c, pl.ds(s * ROWS_PT, ROWS_PT)])


def _prep_body(x_ref, degp_ref, wcat_ref, wz0_ref, wh0_ref, bz_ref, bh_ref,
               y_ref, xz_ref, xh_ref, dinv_ref):
    deg = degp_ref[0, :, 0:1] + degp_ref[1, :, 0:1]
    dinv = jnp.where(deg > 0, lax.rsqrt(jnp.maximum(deg, 1e-12)), 0.0)
    xb = x_ref[...]
    y_ref[...] = jnp.dot(xb * dinv, wcat_ref[...],
                         preferred_element_type=jnp.float32)
    xz_ref[...] = jnp.dot(xb, wz0_ref[...],
                          preferred_element_type=jnp.float32) + bz_ref[...]
    xh_ref[...] = jnp.dot(xb, wh0_ref[...],
                          preferred_element_type=jnp.float32) + bh_ref[...]
    dinv_ref[...] = dinv


def _final_body(aggp_ref, dinv_ref, xz_ref, xh_ref, wlin_ref, blin_ref, out_ref):
    agg = aggp_ref[0] + aggp_ref[1]
    t = agg * (-dinv_ref[...])
    z = jax.nn.sigmoid(xz_ref[...] + t[:, :HID])
    ht = jnp.tanh(xh_ref[...] + t[:, HID:])
    h = jnp.dot((1.0 - z) * ht, wlin_ref[...],
                preferred_element_type=jnp.float32) + blin_ref[...]
    nrm = jnp.maximum(jnp.sqrt(jnp.sum(h * h, axis=1, keepdims=True)), 1e-12)
    out_ref[...] = h / nrm


def kernel(x, edge_index, W_xz, b_xz, W_hz, b_hz, W_xr, b_xr, W_hr, b_hr,
           W_xh, b_xh, W_hh, b_hh, W_lin, b_lin):
    e = edge_index.shape[1]
    cpw = -(-e // (NW * CH))
    e_pad = NW * cpw * CH
    pad = e_pad - e

    src = edge_index[0]
    dst = edge_index[1]
    deg_idx = jnp.concatenate(
        [src, jnp.full((pad,), DUMMY, jnp.int32)]).reshape(NW, cpw, CH)

    e0 = NS * CPW_SC0 * CH
    e1 = NS * CPW_SC1 * CH
    pad1 = e0 + e1 - e

    def slabs(idx, fill):
        a0 = idx[:e0].reshape(NS, CPW_SC0, CH)
        a1 = jnp.concatenate(
            [idx[e0:], jnp.full((pad1,), fill, jnp.int32)]).reshape(
                NS, CPW_SC1, CH)
        out = jnp.full((NW, CPW_MAX, CH), fill, jnp.int32)
        return out.at[:NS, :CPW_SC0].set(a0).at[NS:, :CPW_SC1].set(a1)

    gat_idx = slabs(src, 0)
    sct_idx = slabs(dst, DUMMY)

    ones_rows = jnp.ones((CH, F_IN), jnp.float32)
    zeros_agg = jnp.zeros((ROWS_PT, F_IN), jnp.float32)

    mesh = plsc.VectorSubcoreMesh(core_axis_name="c", subcore_axis_name="s")

    deg_call = pl.kernel(
        _deg_body,
        out_type=jax.ShapeDtypeStruct((NC, N_PAD, F_IN), jnp.float32),
        mesh=mesh,
        scratch_types=[
            pltpu.VMEM((cpw, CH), jnp.int32),
            pltpu.VMEM((CH,), jnp.int32),
            pltpu.VMEM((CH, F_IN), jnp.float32),
            pltpu.VMEM_SHARED((N_PAD, F_IN), jnp.float32),
            pltpu.SemaphoreType.DMA,
        ],
    )
    degp = deg_call(deg_idx, ones_rows, zeros_agg)

    wcat = jnp.concatenate([W_xz[1], W_xh[1]], axis=1)
    bz = (b_xz + b_hz).reshape(1, HID)
    bh = (b_xh + b_hh).reshape(1, HID)

    nblk = N // BLK
    y, xz, xh, dinv = pl.pallas_call(
        _prep_body,
        grid=(nblk,),
        in_specs=[
            pl.BlockSpec((BLK, F_IN), lambda i: (i, 0)),
            pl.BlockSpec((NC, BLK, F_IN), lambda i: (0, i, 0)),
            pl.BlockSpec((F_IN, F_IN), lambda i: (0, 0)),
            pl.BlockSpec((F_IN, HID), lambda i: (0, 0)),
            pl.BlockSpec((F_IN, HID), lambda i: (0, 0)),
            pl.BlockSpec((1, HID), lambda i: (0, 0)),
            pl.BlockSpec((1, HID), lambda i: (0, 0)),
        ],
        out_specs=[
            pl.BlockSpec((BLK, F_IN), lambda i: (i, 0)),
            pl.BlockSpec((BLK, HID), lambda i: (i, 0)),
            pl.BlockSpec((BLK, HID), lambda i: (i, 0)),
            pl.BlockSpec((BLK, 1), lambda i: (i, 0)),
        ],
        out_shape=[
            jax.ShapeDtypeStruct((N, F_IN), jnp.float32),
            jax.ShapeDtypeStruct((N, HID), jnp.float32),
            jax.ShapeDtypeStruct((N, HID), jnp.float32),
            jax.ShapeDtypeStruct((N, 1), jnp.float32),
        ],
    )(x, degp, wcat, W_xz[0], W_xh[0], bz, bh)

    agg_call = pl.kernel(
        _agg_body,
        out_type=jax.ShapeDtypeStruct((NC, N_PAD, F_IN), jnp.float32),
        mesh=mesh,
        scratch_types=[
            pltpu.VMEM((CH,), jnp.int32),
            pltpu.VMEM((CH,), jnp.int32),
            pltpu.VMEM((CH,), jnp.int32),
            pltpu.VMEM((CH,), jnp.int32),
            pltpu.VMEM((CH, F_IN), jnp.float32),
            pltpu.VMEM((CH, F_IN), jnp.float32),
            pltpu.VMEM_SHARED((N_PAD, F_IN), jnp.float32),
            pltpu.SemaphoreType.DMA,
            pltpu.SemaphoreType.DMA,
            pltpu.SemaphoreType.DMA,
            pltpu.SemaphoreType.DMA,
        ],
    )
    aggp = agg_call(y, gat_idx, sct_idx, zeros_agg)

    out = pl.pallas_call(
        _final_body,
        grid=(nblk,),
        in_specs=[
            pl.BlockSpec((NC, BLK, F_IN), lambda i: (0, i, 0)),
            pl.BlockSpec((BLK, 1), lambda i: (i, 0)),
            pl.BlockSpec((BLK, HID), lambda i: (i, 0)),
            pl.BlockSpec((BLK, HID), lambda i: (i, 0)),
            pl.BlockSpec((HID, OUT), lambda i: (0, 0)),
            pl.BlockSpec((1, OUT), lambda i: (0, 0)),
        ],
        out_specs=pl.BlockSpec((BLK, OUT), lambda i: (i, 0)),
        out_shape=jax.ShapeDtypeStruct((N, OUT), jnp.float32),
    )(aggp, dinv, xz, xh, W_lin, b_lin.reshape(1, OUT))
    return out


# speedup vs baseline: 28.5123x; 1.1471x over previous
import functools

import jax
import jax.numpy as jnp
from jax import lax
from jax.experimental import pallas as pl
from jax.experimental.pallas import tpu as pltpu
from jax.experimental.pallas import tpu_sc as plsc

N = 10000
F_IN = 128
HID = 64
OUT = 16

NC = 2
NS = 16
NW = NC * NS
CH = 128

N_PAD = 10240
ROWS_PT = N_PAD // NS
DUMMY = N

BLK = 1000

CPW_SC0 = 90
CPW_SC1 = 67
CPW_MAX = max(CPW_SC0, CPW_SC1)


def _deg_body(didx_hbm, ones_hbm, zeros_hbm, out_hbm, idx_v, idx_cur, ones_v,
              acc_sh, sem):
    c = lax.axis_index("c")
    s = lax.axis_index("s")
    w = c * NS + s
    cpw = didx_hbm.shape[1]
    pltpu.sync_copy(didx_hbm.at[w], idx_v)
    pltpu.sync_copy(ones_hbm, ones_v)
    pltpu.sync_copy(zeros_hbm, acc_sh.at[pl.ds(s * ROWS_PT, ROWS_PT)])
    plsc.subcore_barrier()

    def body(j, carry):
        for g in range(CH // 16):
            idx_cur[pl.ds(g * 16, 16)] = idx_v[j, pl.ds(g * 16, 16)]
        pltpu.sync_copy(ones_v, acc_sh.at[idx_cur], add=True)
        return carry

    lax.fori_loop(0, cpw, body, 0)
    plsc.subcore_barrier()
    pltpu.sync_copy(acc_sh.at[pl.ds(s * ROWS_PT, ROWS_PT)],
                    out_hbm.at[c, pl.ds(s * ROWS_PT, ROWS_PT)])


def _agg_body(y_hbm, gidx_hbm, sidx_hbm, zeros_hbm, out_hbm,
              gcur0, scur0, gcur1, scur1, rows0, rows1,
              acc_sh, semA, semB, semI0, semI1):
    c = lax.axis_index("c")
    s = lax.axis_index("s")
    w = c * NS + s
    cpw = jnp.where(c == 0, CPW_SC0, CPW_SC1)
    pltpu.sync_copy(zeros_hbm, acc_sh.at[pl.ds(s * ROWS_PT, ROWS_PT)])

    def fetch(j, gbuf, sbuf, sem):
        pltpu.async_copy(gidx_hbm.at[w, j], gbuf, sem)
        pltpu.async_copy(sidx_hbm.at[w, j], sbuf, sem)

    def wait_fetch(gbuf, sbuf, sem):
        pltpu.make_async_copy(gidx_hbm.at[w, 0], gbuf, sem).wait()
        pltpu.make_async_copy(sidx_hbm.at[w, 0], sbuf, sem).wait()

    plsc.subcore_barrier()

    fetch(0, gcur0, scur0, semI0)
    wait_fetch(gcur0, scur0, semI0)
    pltpu.async_copy(y_hbm.at[gcur0], rows0, semA)
    fetch(1, gcur1, scur1, semI1)

    def body(jj, carry):
        a = 2 * jj
        b = a + 1
        wait_fetch(gcur1, scur1, semI1)
        pltpu.async_copy(y_hbm.at[gcur1], rows1, semB)
        pltpu.make_async_copy(y_hbm.at[gcur0], rows0, semA).wait()

        @pl.when(a + 2 < cpw)
        def _():
            pltpu.async_copy(gidx_hbm.at[w, a + 2], gcur0, semI0)

        pltpu.sync_copy(rows0, acc_sh.at[scur0], add=True)

        @pl.when(a + 2 < cpw)
        def _():
            pltpu.async_copy(sidx_hbm.at[w, a + 2], scur0, semI0)
            wait_fetch(gcur0, scur0, semI0)
            pltpu.async_copy(y_hbm.at[gcur0], rows0, semA)

        pltpu.make_async_copy(y_hbm.at[gcur1], rows1, semB).wait()
        pltpu.sync_copy(rows1, acc_sh.at[scur1], add=True)

        @pl.when(b + 2 < cpw)
        def _():
            fetch(b + 2, gcur1, scur1, semI1)

        return carry

    lax.fori_loop(0, cpw // 2, body, 0)

    @pl.when(cpw % 2 == 1)
    def _():
        pltpu.make_async_copy(y_hbm.at[gcur0], rows0, semA).wait()
        pltpu.sync_copy(rows0, acc_sh.at[scur0], add=True)
    plsc.subcore_barrier()
    pltpu.sync_copy(acc_sh.at[pl.ds(s * ROWS_PT, ROWS_PT)],
                    out_hbm.at[c, pl.ds(s * ROWS_PT, ROWS_PT)])


def _prep_body(x_ref, degp_ref, wcat_ref, wz0_ref, wh0_ref, bz_ref, bh_ref,
               y_ref, xz_ref, xh_ref, dinv_ref):
    deg = degp_ref[0, :, 0:1] + degp_ref[1, :, 0:1]
    dinv = jnp.where(deg > 0, lax.rsqrt(jnp.maximum(deg, 1e-12)), 0.0)
    xb = x_ref[...]
    y_ref[...] = jnp.dot(xb * dinv, wcat_ref[...],
                         preferred_element_type=jnp.float32)
    xz_ref[...] = jnp.dot(xb, wz0_ref[...],
                          preferred_element_type=jnp.float32) + bz_ref[...]
    xh_ref[...] = jnp.dot(xb, wh0_ref[...],
                          preferred_element_type=jnp.float32) + bh_ref[...]
    dinv_ref[...] = dinv


def _final_body(aggp_ref, dinv_ref, xz_ref, xh_ref, wlin_ref, blin_ref, out_ref):
    agg = aggp_ref[0] + aggp_ref[1]
    t = agg * (-dinv_ref[...])
    z = jax.nn.sigmoid(xz_ref[...] + t[:, :HID])
    ht = jnp.tanh(xh_ref[...] + t[:, HID:])
    h = jnp.dot((1.0 - z) * ht, wlin_ref[...],
                preferred_element_type=jnp.float32) + blin_ref[...]
    nrm = jnp.maximum(jnp.sqrt(jnp.sum(h * h, axis=1, keepdims=True)), 1e-12)
    out_ref[...] = h / nrm


def kernel(x, edge_index, W_xz, b_xz, W_hz, b_hz, W_xr, b_xr, W_hr, b_hr,
           W_xh, b_xh, W_hh, b_hh, W_lin, b_lin):
    e = edge_index.shape[1]
    cpw = -(-e // (NW * CH))
    e_pad = NW * cpw * CH
    pad = e_pad - e

    src = edge_index[0]
    dst = edge_index[1]
    deg_idx = jnp.concatenate(
        [src, jnp.full((pad,), DUMMY, jnp.int32)]).reshape(NW, cpw, CH)

    e0 = NS * CPW_SC0 * CH
    e1 = NS * CPW_SC1 * CH
    pad1 = e0 + e1 - e

    def slabs(idx, fill):
        a0 = idx[:e0].reshape(NS, CPW_SC0, CH)
        a1 = jnp.concatenate(
            [idx[e0:], jnp.full((pad1,), fill, jnp.int32)]).reshape(
                NS, CPW_SC1, CH)
        out = jnp.full((NW, CPW_MAX, CH), fill, jnp.int32)
        return out.at[:NS, :CPW_SC0].set(a0).at[NS:, :CPW_SC1].set(a1)

    gat_idx = slabs(src, 0)
    sct_idx = slabs(dst, DUMMY)

    ones_rows = jnp.ones((CH, F_IN), jnp.float32)
    zeros_agg = jnp.zeros((ROWS_PT, F_IN), jnp.float32)

    mesh = plsc.VectorSubcoreMesh(core_axis_name="c", subcore_axis_name="s")

    deg_call = pl.kernel(
        _deg_body,
        out_type=jax.ShapeDtypeStruct((NC, N_PAD, F_IN), jnp.float32),
        mesh=mesh,
        scratch_types=[
            pltpu.VMEM((cpw, CH), jnp.int32),
            pltpu.VMEM((CH,), jnp.int32),
            pltpu.VMEM((CH, F_IN), jnp.float32),
            pltpu.VMEM_SHARED((N_PAD, F_IN), jnp.float32),
            pltpu.SemaphoreType.DMA,
        ],
    )
    degp = deg_call(deg_idx, ones_rows, zeros_agg)

    wcat = jnp.concatenate([W_xz[1], W_xh[1]], axis=1)
    bz = (b_xz + b_hz).reshape(1, HID)
    bh = (b_xh + b_hh).reshape(1, HID)

    nblk = N // BLK
    y, xz, xh, dinv = pl.pallas_call(
        _prep_body,
        grid=(nblk,),
        in_specs=[
            pl.BlockSpec((BLK, F_IN), lambda i: (i, 0)),
            pl.BlockSpec((NC, BLK, F_IN), lambda i: (0, i, 0)),
            pl.BlockSpec((F_IN, F_IN), lambda i: (0, 0)),
            pl.BlockSpec((F_IN, HID), lambda i: (0, 0)),
            pl.BlockSpec((F_IN, HID), lambda i: (0, 0)),
            pl.BlockSpec((1, HID), lambda i: (0, 0)),
            pl.BlockSpec((1, HID), lambda i: (0, 0)),
        ],
        out_specs=[
            pl.BlockSpec((BLK, F_IN), lambda i: (i, 0)),
            pl.BlockSpec((BLK, HID), lambda i: (i, 0)),
            pl.BlockSpec((BLK, HID), lambda i: (i, 0)),
            pl.BlockSpec((BLK, 1), lambda i: (i, 0)),
        ],
        out_shape=[
            jax.ShapeDtypeStruct((N, F_IN), jnp.float32),
            jax.ShapeDtypeStruct((N, HID), jnp.float32),
            jax.ShapeDtypeStruct((N, HID), jnp.float32),
            jax.ShapeDtypeStruct((N, 1), jnp.float32),
        ],
    )(x, degp, wcat, W_xz[0], W_xh[0], bz, bh)

    agg_call = pl.kernel(
        _agg_body,
        out_type=jax.ShapeDtypeStruct((NC, N_PAD, F_IN), jnp.float32),
        mesh=mesh,
        scratch_types=[
            pltpu.VMEM((CH,), jnp.int32),
            pltpu.VMEM((CH,), jnp.int32),
            pltpu.VMEM((CH,), jnp.int32),
            pltpu.VMEM((CH,), jnp.int32),
            pltpu.VMEM((CH, F_IN), jnp.float32),
            pltpu.VMEM((CH, F_IN), jnp.float32),
            pltpu.VMEM_SHARED((N_PAD, F_IN), jnp.float32),
            pltpu.SemaphoreType.DMA,
            pltpu.SemaphoreType.DMA,
            pltpu.SemaphoreType.DMA,
            pltpu.SemaphoreType.DMA,
        ],
    )
    aggp = agg_call(y, gat_idx, sct_idx, zeros_agg)

    out = pl.pallas_call(
        _final_body,
        grid=(nblk,),
        in_specs=[
            pl.BlockSpec((NC, BLK, F_IN), lambda i: (0, i, 0)),
            pl.BlockSpec((BLK, 1), lambda i: (i, 0)),
            pl.BlockSpec((BLK, HID), lambda i: (i, 0)),
            pl.BlockSpec((BLK, HID), lambda i: (i, 0)),
            pl.BlockSpec((HID, OUT), lambda i: (0, 0)),
            pl.BlockSpec((1, OUT), lambda i: (0, 0)),
        ],
        out_specs=pl.BlockSpec((BLK, OUT), lambda i: (i, 0)),
        out_shape=jax.ShapeDtypeStruct((N, OUT), jnp.float32),
    )(aggp, dinv, xz, xh, W_lin, b_lin.reshape(1, OUT))
    return out


# speedup vs baseline: 30.1689x; 1.0581x over previous
import functools

import jax
import jax.numpy as jnp
from jax import lax
from jax.experimental import pallas as pl
from jax.experimental.pallas import tpu as pltpu
from jax.experimental.pallas import tpu_sc as plsc

N = 10000
F_IN = 128
HID = 64
OUT = 16

NC = 2
NS = 16
NW = NC * NS
CH = 128

N_PAD = 10240
ROWS_PT = N_PAD // NS
DUMMY = N

BLK = 1000

CPW_SC0 = 84
CPW_SC1 = 73
CPW_MAX = max(CPW_SC0, CPW_SC1)


def _deg_body(didx_hbm, ones_hbm, zeros_hbm, out_hbm, idx_v, idx_cur, ones_v,
              acc_sh, sem):
    c = lax.axis_index("c")
    s = lax.axis_index("s")
    w = c * NS + s
    cpw = didx_hbm.shape[1]
    pltpu.sync_copy(didx_hbm.at[w], idx_v)
    pltpu.sync_copy(ones_hbm, ones_v)
    pltpu.sync_copy(zeros_hbm, acc_sh.at[pl.ds(s * ROWS_PT, ROWS_PT)])
    plsc.subcore_barrier()

    def body(j, carry):
        for g in range(CH // 16):
            idx_cur[pl.ds(g * 16, 16)] = idx_v[j, pl.ds(g * 16, 16)]
        pltpu.sync_copy(ones_v, acc_sh.at[idx_cur], add=True)
        return carry

    lax.fori_loop(0, cpw, body, 0)
    plsc.subcore_barrier()
    pltpu.sync_copy(acc_sh.at[pl.ds(s * ROWS_PT, ROWS_PT)],
                    out_hbm.at[c, pl.ds(s * ROWS_PT, ROWS_PT)])


def _agg_body(y_hbm, gidx_hbm, sidx_hbm, zeros_hbm, out_hbm,
              gcur0, scur0, gcur1, scur1, rows0, rows1,
              acc_sh, semA, semB, semI0, semI1):
    c = lax.axis_index("c")
    s = lax.axis_index("s")
    w = c * NS + s
    cpw = jnp.where(c == 0, CPW_SC0, CPW_SC1)
    pltpu.sync_copy(zeros_hbm, acc_sh.at[pl.ds(s * ROWS_PT, ROWS_PT)])

    def fetch(j, gbuf, sbuf, sem):
        pltpu.async_copy(gidx_hbm.at[w, j], gbuf, sem)
        pltpu.async_copy(sidx_hbm.at[w, j], sbuf, sem)

    def wait_fetch(gbuf, sbuf, sem):
        pltpu.make_async_copy(gidx_hbm.at[w, 0], gbuf, sem).wait()
        pltpu.make_async_copy(sidx_hbm.at[w, 0], sbuf, sem).wait()

    plsc.subcore_barrier()

    fetch(0, gcur0, scur0, semI0)
    wait_fetch(gcur0, scur0, semI0)
    pltpu.async_copy(y_hbm.at[gcur0], rows0, semA)
    fetch(1, gcur1, scur1, semI1)

    def body(jj, carry):
        a = 2 * jj
        b = a + 1
        wait_fetch(gcur1, scur1, semI1)
        pltpu.async_copy(y_hbm.at[gcur1], rows1, semB)
        pltpu.make_async_copy(y_hbm.at[gcur0], rows0, semA).wait()

        @pl.when(a + 2 < cpw)
        def _():
            pltpu.async_copy(gidx_hbm.at[w, a + 2], gcur0, semI0)

        pltpu.sync_copy(rows0, acc_sh.at[scur0], add=True)

        @pl.when(a + 2 < cpw)
        def _():
            pltpu.async_copy(sidx_hbm.at[w, a + 2], scur0, semI0)
            wait_fetch(gcur0, scur0, semI0)
            pltpu.async_copy(y_hbm.at[gcur0], rows0, semA)

        pltpu.make_async_copy(y_hbm.at[gcur1], rows1, semB).wait()
        pltpu.sync_copy(rows1, acc_sh.at[scur1], add=True)

        @pl.when(b + 2 < cpw)
        def _():
            fetch(b + 2, gcur1, scur1, semI1)

        return carry

    lax.fori_loop(0, cpw // 2, body, 0)

    @pl.when(cpw % 2 == 1)
    def _():
        pltpu.make_async_copy(y_hbm.at[gcur0], rows0, semA).wait()
        pltpu.sync_copy(rows0, acc_sh.at[scur0], add=True)
    plsc.subcore_barrier()
    pltpu.sync_copy(acc_sh.at[pl.ds(s * ROWS_PT, ROWS_PT)],
                    out_hbm.at[c, pl.ds(s * ROWS_PT, ROWS_PT)])


def _prep_body(x_ref, degp_ref, wcat_ref, wz0_ref, wh0_ref, bz_ref, bh_ref,
               y_ref, xz_ref, xh_ref, dinv_ref):
    deg = degp_ref[0, :, 0:1] + degp_ref[1, :, 0:1]
    dinv = jnp.where(deg > 0, lax.rsqrt(jnp.maximum(deg, 1e-12)), 0.0)
    xb = x_ref[...]
    y_ref[...] = jnp.dot(xb * dinv, wcat_ref[...],
                         preferred_element_type=jnp.float32)
    xz_ref[...] = jnp.dot(xb, wz0_ref[...],
                          preferred_element_type=jnp.float32) + bz_ref[...]
    xh_ref[...] = jnp.dot(xb, wh0_ref[...],
                          preferred_element_type=jnp.float32) + bh_ref[...]
    dinv_ref[...] = dinv


def _final_body(aggp_ref, dinv_ref, xz_ref, xh_ref, wlin_ref, blin_ref, out_ref):
    agg = aggp_ref[0] + aggp_ref[1]
    t = agg * (-dinv_ref[...])
    z = jax.nn.sigmoid(xz_ref[...] + t[:, :HID])
    ht = jnp.tanh(xh_ref[...] + t[:, HID:])
    h = jnp.dot((1.0 - z) * ht, wlin_ref[...],
                preferred_element_type=jnp.float32) + blin_ref[...]
    nrm = jnp.maximum(jnp.sqrt(jnp.sum(h * h, axis=1, keepdims=True)), 1e-12)
    out_ref[...] = h / nrm


def kernel(x, edge_index, W_xz, b_xz, W_hz, b_hz, W_xr, b_xr, W_hr, b_hr,
           W_xh, b_xh, W_hh, b_hh, W_lin, b_lin):
    e = edge_index.shape[1]
    cpw = -(-e // (NW * CH))
    e_pad = NW * cpw * CH
    pad = e_pad - e

    src = edge_index[0]
    dst = edge_index[1]
    deg_idx = jnp.concatenate(
        [src, jnp.full((pad,), DUMMY, jnp.int32)]).reshape(NW, cpw, CH)

    e0 = NS * CPW_SC0 * CH
    e1 = NS * CPW_SC1 * CH
    pad1 = e0 + e1 - e

    def slabs(idx, fill):
        a0 = idx[:e0].reshape(NS, CPW_SC0, CH)
        a1 = jnp.concatenate(
            [idx[e0:], jnp.full((pad1,), fill, jnp.int32)]).reshape(
                NS, CPW_SC1, CH)
        out = jnp.full((NW, CPW_MAX, CH), fill, jnp.int32)
        return out.at[:NS, :CPW_SC0].set(a0).at[NS:, :CPW_SC1].set(a1)

    gat_idx = slabs(src, 0)
    sct_idx = slabs(dst, DUMMY)

    ones_rows = jnp.ones((CH, F_IN), jnp.float32)
    zeros_agg = jnp.zeros((ROWS_PT, F_IN), jnp.float32)

    mesh = plsc.VectorSubcoreMesh(core_axis_name="c", subcore_axis_name="s")

    deg_call = pl.kernel(
        _deg_body,
        out_type=jax.ShapeDtypeStruct((NC, N_PAD, F_IN), jnp.float32),
        mesh=mesh,
        scratch_types=[
            pltpu.VMEM((cpw, CH), jnp.int32),
            pltpu.VMEM((CH,), jnp.int32),
            pltpu.VMEM((CH, F_IN), jnp.float32),
            pltpu.VMEM_SHARED((N_PAD, F_IN), jnp.float32),
            pltpu.SemaphoreType.DMA,
        ],
    )
    degp = deg_call(deg_idx, ones_rows, zeros_agg)

    wcat = jnp.concatenate([W_xz[1], W_xh[1]], axis=1)
    bz = (b_xz + b_hz).reshape(1, HID)
    bh = (b_xh + b_hh).reshape(1, HID)

    nblk = N // BLK
    y, xz, xh, dinv = pl.pallas_call(
        _prep_body,
        grid=(nblk,),
        in_specs=[
            pl.BlockSpec((BLK, F_IN), lambda i: (i, 0)),
            pl.BlockSpec((NC, BLK, F_IN), lambda i: (0, i, 0)),
            pl.BlockSpec((F_IN, F_IN), lambda i: (0, 0)),
            pl.BlockSpec((F_IN, HID), lambda i: (0, 0)),
            pl.BlockSpec((F_IN, HID), lambda i: (0, 0)),
            pl.BlockSpec((1, HID), lambda i: (0, 0)),
            pl.BlockSpec((1, HID), lambda i: (0, 0)),
        ],
        out_specs=[
            pl.BlockSpec((BLK, F_IN), lambda i: (i, 0)),
            pl.BlockSpec((BLK, HID), lambda i: (i, 0)),
            pl.BlockSpec((BLK, HID), lambda i: (i, 0)),
            pl.BlockSpec((BLK, 1), lambda i: (i, 0)),
        ],
        out_shape=[
            jax.ShapeDtypeStruct((N, F_IN), jnp.float32),
            jax.ShapeDtypeStruct((N, HID), jnp.float32),
            jax.ShapeDtypeStruct((N, HID), jnp.float32),
            jax.ShapeDtypeStruct((N, 1), jnp.float32),
        ],
    )(x, degp, wcat, W_xz[0], W_xh[0], bz, bh)

    agg_call = pl.kernel(
        _agg_body,
        out_type=jax.ShapeDtypeStruct((NC, N_PAD, F_IN), jnp.float32),
        mesh=mesh,
        scratch_types=[
            pltpu.VMEM((CH,), jnp.int32),
            pltpu.VMEM((CH,), jnp.int32),
            pltpu.VMEM((CH,), jnp.int32),
            pltpu.VMEM((CH,), jnp.int32),
            pltpu.VMEM((CH, F_IN), jnp.float32),
            pltpu.VMEM((CH, F_IN), jnp.float32),
            pltpu.VMEM_SHARED((N_PAD, F_IN), jnp.float32),
            pltpu.SemaphoreType.DMA,
            pltpu.SemaphoreType.DMA,
            pltpu.SemaphoreType.DMA,
            pltpu.SemaphoreType.DMA,
        ],
    )
    aggp = agg_call(y, gat_idx, sct_idx, zeros_agg)

    out = pl.pallas_call(
        _final_body,
        grid=(nblk,),
        in_specs=[
            pl.BlockSpec((NC, BLK, F_IN), lambda i: (0, i, 0)),
            pl.BlockSpec((BLK, 1), lambda i: (i, 0)),
            pl.BlockSpec((BLK, HID), lambda i: (i, 0)),
            pl.BlockSpec((BLK, HID), lambda i: (i, 0)),
            pl.BlockSpec((HID, OUT), lambda i: (0, 0)),
            pl.BlockSpec((1, OUT), lambda i: (0, 0)),
        ],
        out_specs=pl.BlockSpec((BLK, OUT), lambda i: (i, 0)),
        out_shape=jax.ShapeDtypeStruct((N, OUT), jnp.float32),
    )(aggp, dinv, xz, xh, W_lin, b_lin.reshape(1, OUT))
    return out
